# Initial kernel scaffold; baseline (speedup 1.0000x reference)
import jax
import jax.numpy as jnp
from jax.experimental import pallas as pl


def kernel(price_input, e, concept, Wih, Whh, bih, bhh, W2, b2, Wh1, bh1, Wh2, bh2, W1, b1):
    raise NotImplementedError("write your pallas kernel here")


# speedup vs baseline: 18.1064x; 18.1064x over previous
import functools

import jax
import jax.numpy as jnp
from jax import lax
from jax.experimental import pallas as pl
from jax.experimental.pallas import tpu as pltpu
from jax.experimental.pallas import tpu_sc as plsc

_N = 50000
_F = 32
_T = 20
_NNZ = 1600000
_C = 128
_NC = 2
_NS = 16
_NW = _NC * _NS
_CHUNKS = 12544
_NNZ_P = _CHUNKS * _C
_CPT = _CHUNKS // _NW
_G = 8
_NP = 51200
_RPT = _NP // _NS
_PAD_IDX = _N
_BN = 2048
_RB = 6400

_mesh = plsc.VectorSubcoreMesh(
    core_axis_name="c", subcore_axis_name="s", num_cores=_NC, num_subcores=_NS
)


def _build_sc_pass(with_counts):
    outs = [jax.ShapeDtypeStruct((_NC, _NP, _F), jnp.float32)]
    if with_counts:
        outs += [jax.ShapeDtypeStruct((_NC, _NP), jnp.float32)] * 2
    scratch = [
        pltpu.VMEM_SHARED((_NP, _F), jnp.float32),
        pltpu.VMEM((_G, _C), jnp.int32),
        pltpu.VMEM((_G, _C), jnp.int32),
        pltpu.VMEM((_C, _F), jnp.float32),
        pltpu.SemaphoreType.DMA,
    ]
    if with_counts:
        scratch += [
            pltpu.VMEM_SHARED((_NP,), jnp.float32),
            pltpu.VMEM_SHARED((_NP,), jnp.float32),
            pltpu.VMEM((_C,), jnp.float32),
        ]

    def body(*refs):
        if with_counts:
            (x_hbm, src_hbm, dst_hbm, z2_hbm, z1_hbm, ones_hbm,
             part_hbm, dcnt_hbm, bcnt_hbm,
             acc, src_v, dst_v, rows_v, sem, dacc, bacc, ones_v) = refs
        else:
            (x_hbm, src_hbm, dst_hbm, z2_hbm,
             part_hbm, acc, src_v, dst_v, rows_v, sem) = refs
        c = lax.axis_index("c")
        s = lax.axis_index("s")
        wid = s * _NC + c
        rbase = s * _RPT
        pltpu.sync_copy(z2_hbm, acc.at[pl.ds(rbase, _RPT), :])
        if with_counts:
            pltpu.sync_copy(z1_hbm, dacc.at[pl.ds(rbase, _RPT)])
            pltpu.sync_copy(z1_hbm, bacc.at[pl.ds(rbase, _RPT)])
            pltpu.sync_copy(ones_hbm, ones_v)
        plsc.subcore_barrier()

        def group(g, carry):
            cbase = wid * _CPT + g * _G
            pltpu.sync_copy(src_hbm.at[pl.ds(cbase, _G), :], src_v)
            pltpu.sync_copy(dst_hbm.at[pl.ds(cbase, _G), :], dst_v)

            def step(j, cc):
                pltpu.async_copy(x_hbm.at[src_v.at[j]], rows_v, sem).wait()
                pltpu.sync_copy(rows_v, acc.at[dst_v.at[j]], add=True)
                if with_counts:
                    pltpu.sync_copy(ones_v, dacc.at[src_v.at[j]], add=True)
                    pltpu.sync_copy(ones_v, bacc.at[dst_v.at[j]], add=True)
                return cc

            lax.fori_loop(0, _G, step, 0)
            return carry

        lax.fori_loop(0, _CPT // _G, group, 0)
        plsc.subcore_barrier()
        pltpu.sync_copy(acc.at[pl.ds(rbase, _RPT), :],
                        part_hbm.at[c, pl.ds(rbase, _RPT), :])
        if with_counts:
            pltpu.sync_copy(dacc.at[pl.ds(rbase, _RPT)],
                            dcnt_hbm.at[c, pl.ds(rbase, _RPT)])
            pltpu.sync_copy(bacc.at[pl.ds(rbase, _RPT)],
                            bcnt_hbm.at[c, pl.ds(rbase, _RPT)])

    return pl.kernel(body, out_type=tuple(outs) if with_counts else outs[0],
                     mesh=_mesh, scratch_types=scratch,
                     compiler_params=pltpu.CompilerParams(
                         use_tc_tiling_on_sc=False))


_sc_pass_counts = _build_sc_pass(True)
_sc_pass = _build_sc_pass(False)


def _leaky(x, a):
    return jnp.where(x >= 0, x, a * x)


def _front_body(x_ref, wih_ref, whh_ref, bih_ref, bhh_ref, w2_ref, b2_ref,
                wh1_ref, o_ref):
    x = x_ref[...]
    h = jnp.zeros((_BN, _F), jnp.float32)
    acc = jnp.zeros((_BN, _F), jnp.float32)
    for t in range(_T):
        xt = x[t]
        ir = xt @ wih_ref[0] + bih_ref[0]
        iz = xt @ wih_ref[1] + bih_ref[1]
        inn = xt @ wih_ref[2] + bih_ref[2]
        hr = h @ whh_ref[0] + bhh_ref[0]
        hz = h @ whh_ref[1] + bhh_ref[1]
        hn = h @ whh_ref[2] + bhh_ref[2]
        r = jax.nn.sigmoid(ir + hr)
        z = jax.nn.sigmoid(iz + hz)
        nn = jnp.tanh(inn + r * hn)
        h = (1.0 - z) * nn + z * h
        acc = acc + h @ w2_ref[t]
    out = _leaky(acc + b2_ref[...], 0.01)
    o_ref[...] = out @ wh1_ref[...]


_front = pl.pallas_call(
    _front_body,
    grid=(_NP // _BN,),
    in_specs=[
        pl.BlockSpec((_T, _BN, 6), lambda i: (0, i, 0)),
        pl.BlockSpec((3, 6, _F), lambda i: (0, 0, 0)),
        pl.BlockSpec((3, _F, _F), lambda i: (0, 0, 0)),
        pl.BlockSpec((3, 1, _F), lambda i: (0, 0, 0)),
        pl.BlockSpec((3, 1, _F), lambda i: (0, 0, 0)),
        pl.BlockSpec((_T, _F, _F), lambda i: (0, 0, 0)),
        pl.BlockSpec((1, _F), lambda i: (0, 0)),
        pl.BlockSpec((_F, _F), lambda i: (0, 0)),
    ],
    out_specs=pl.BlockSpec((_BN, _F), lambda i: (i, 0)),
    out_shape=jax.ShapeDtypeStruct((_NP, _F), jnp.float32),
)


def _combe_body(p_ref, b_ref, o_ref):
    sacc = p_ref[0] + p_ref[1]
    bc = b_ref[0] + b_ref[1]
    binv = jnp.where(bc > 0, 1.0 / bc, 0.0)
    o_ref[...] = binv * sacc


_combE = pl.pallas_call(
    _combe_body,
    grid=(_NP // _RB,),
    in_specs=[
        pl.BlockSpec((_NC, _RB, _F), lambda i: (0, i, 0)),
        pl.BlockSpec((_NC, _RB, 1), lambda i: (0, i, 0)),
    ],
    out_specs=pl.BlockSpec((_RB, _F), lambda i: (i, 0)),
    out_shape=jax.ShapeDtypeStruct((_NP, _F), jnp.float32),
)


def _build_combn(final):
    def body(*refs):
        if final:
            p_ref, d_ref, bh_ref, w_ref, b1_ref, o_ref = refs
        else:
            p_ref, d_ref, bh_ref, w_ref, o_ref = refs
        sacc = p_ref[0] + p_ref[1]
        dc = d_ref[0] + d_ref[1]
        dinv = jnp.where(dc > 0, 1.0 / dc, 0.0)
        xv = _leaky(dinv * sacc + bh_ref[...], 0.2)
        y = xv @ w_ref[...]
        if final:
            y = _leaky(y + b1_ref[...], 0.01)
        o_ref[...] = y

    in_specs = [
        pl.BlockSpec((_NC, _RB, _F), lambda i: (0, i, 0)),
        pl.BlockSpec((_NC, _RB, 1), lambda i: (0, i, 0)),
        pl.BlockSpec((1, _F), lambda i: (0, 0)),
        pl.BlockSpec((_F, _F), lambda i: (0, 0)),
    ]
    if final:
        in_specs.append(pl.BlockSpec((1, _F), lambda i: (0, 0)))
    return pl.pallas_call(
        body,
        grid=(_NP // _RB,),
        in_specs=in_specs,
        out_specs=pl.BlockSpec((_RB, _F), lambda i: (i, 0)),
        out_shape=jax.ShapeDtypeStruct((_NP, _F), jnp.float32),
    )


_combN = _build_combn(False)
_final = _build_combn(True)


def kernel(price_input, e, concept, Wih, Whh, bih, bhh, W2, b2, Wh1, bh1,
           Wh2, bh2, W1, b1):
    f32 = jnp.float32
    node_idx = e[0]
    edge_idx = e[1]
    pad = _NNZ_P - _NNZ
    padv = jnp.full((pad,), _PAD_IDX, jnp.int32)
    node2d = jnp.concatenate([node_idx, padv]).reshape(_CHUNKS, _C)
    edge2d = jnp.concatenate([edge_idx, padv]).reshape(_CHUNKS, _C)
    z2 = jnp.zeros((_RPT, _F), f32)
    z1 = jnp.zeros((_RPT,), f32)
    ones = jnp.ones((_C,), f32)

    xp = jnp.transpose(price_input, (1, 0, 2))
    xp = jnp.pad(xp, ((0, 0), (0, _NP - _N), (0, 0)))
    wih3 = jnp.transpose(Wih.reshape(3, _F, 6), (0, 2, 1))
    whh3 = jnp.transpose(Whh.reshape(3, _F, _F), (0, 2, 1))
    bih3 = bih.reshape(3, 1, _F)
    bhh3 = bhh.reshape(3, 1, _F)
    w2t = jnp.transpose(W2).reshape(_T, _F, _F)
    b2r = b2.reshape(1, _F)
    wh1t = jnp.transpose(Wh1)
    wh2t = jnp.transpose(Wh2)
    w1t = jnp.transpose(W1)
    bh1r = bh1.reshape(1, _F)
    bh2r = bh2.reshape(1, _F)
    b1r = b1.reshape(1, _F)

    xw1 = _front(xp, wih3, whh3, bih3, bhh3, w2t, b2r, wh1t)

    pA1, dcnt, bcnt = _sc_pass_counts(xw1, node2d, edge2d, z2, z1, ones)
    dcnt3 = dcnt.reshape(_NC, _NP, 1)
    bcnt3 = bcnt.reshape(_NC, _NP, 1)

    ef1 = _combE(pA1, bcnt3)
    pB1 = _sc_pass(ef1, edge2d, node2d, z2)
    xw2 = _combN(pB1, dcnt3, bh1r, wh2t)

    pA2 = _sc_pass(xw2, node2d, edge2d, z2)
    ef2 = _combE(pA2, bcnt3)
    pB2 = _sc_pass(ef2, edge2d, node2d, z2)
    out = _final(pB2, dcnt3, bh2r, w1t, b1r)
    return out[:_N]


# speedup vs baseline: 19.4842x; 1.0761x over previous
import functools

import jax
import jax.numpy as jnp
from jax import lax
from jax.experimental import pallas as pl
from jax.experimental.pallas import tpu as pltpu
from jax.experimental.pallas import tpu_sc as plsc

_N = 50000
_F = 32
_T = 20
_NNZ = 1600000
_C = 128
_NC = 2
_NS = 16
_NW = _NC * _NS
_CHUNKS = 12544
_NNZ_P = _CHUNKS * _C
_CPT = _CHUNKS // _NW
_G = 8
_NP = 51200
_RPT = _NP // _NS
_PAD_IDX = _N
_BN = 2048
_RB = 6400

_mesh = plsc.VectorSubcoreMesh(
    core_axis_name="c", subcore_axis_name="s", num_cores=_NC, num_subcores=_NS
)


def _build_sc_pass(with_counts):
    outs = [jax.ShapeDtypeStruct((_NC, _NP, _F), jnp.float32)]
    if with_counts:
        outs += [jax.ShapeDtypeStruct((_NC, _NP), jnp.float32)] * 2
    scratch = [
        pltpu.VMEM_SHARED((_NP, _F), jnp.float32),
        pltpu.VMEM((_G, _C), jnp.int32),
        pltpu.VMEM((_G, _C), jnp.int32),
        pltpu.VMEM((2, _C, _F), jnp.float32),
        pltpu.SemaphoreType.DMA,
        pltpu.SemaphoreType.DMA,
        pltpu.SemaphoreType.DMA,
    ]
    if with_counts:
        scratch += [
            pltpu.VMEM_SHARED((_NP,), jnp.float32),
            pltpu.VMEM_SHARED((_NP,), jnp.float32),
            pltpu.VMEM((_C,), jnp.float32),
        ]

    def body(*refs):
        if with_counts:
            (x_hbm, src_hbm, dst_hbm, z2_hbm, z1_hbm, ones_hbm,
             part_hbm, dcnt_hbm, bcnt_hbm,
             acc, src_v, dst_v, rows_v, gsem, ssem0, ssem1,
             dacc, bacc, ones_v) = refs
        else:
            (x_hbm, src_hbm, dst_hbm, z2_hbm,
             part_hbm, acc, src_v, dst_v, rows_v, gsem, ssem0, ssem1) = refs
        ssems = (ssem0, ssem1)
        c = lax.axis_index("c")
        s = lax.axis_index("s")
        wid = s * _NC + c
        rbase = s * _RPT
        pltpu.sync_copy(z2_hbm, acc.at[pl.ds(rbase, _RPT), :])
        if with_counts:
            pltpu.sync_copy(z1_hbm, dacc.at[pl.ds(rbase, _RPT)])
            pltpu.sync_copy(z1_hbm, bacc.at[pl.ds(rbase, _RPT)])
            pltpu.sync_copy(ones_hbm, ones_v)
        plsc.subcore_barrier()

        def group(g, carry):
            cbase = wid * _CPT + g * _G
            pltpu.sync_copy(src_hbm.at[pl.ds(cbase, _G), :], src_v)
            pltpu.sync_copy(dst_hbm.at[pl.ds(cbase, _G), :], dst_v)
            descs = {}
            for j in range(_G):
                b = j % 2
                if j >= 2:
                    descs[b].wait()
                pltpu.async_copy(x_hbm.at[src_v.at[j]], rows_v.at[b],
                                 gsem).wait()
                descs[b] = pltpu.async_copy(rows_v.at[b],
                                            acc.at[dst_v.at[j]],
                                            ssems[b], add=True)
                if with_counts:
                    pltpu.sync_copy(ones_v, dacc.at[src_v.at[j]], add=True)
                    pltpu.sync_copy(ones_v, bacc.at[dst_v.at[j]], add=True)
            descs[0].wait()
            descs[1].wait()
            return carry

        lax.fori_loop(0, _CPT // _G, group, 0)
        plsc.subcore_barrier()
        pltpu.sync_copy(acc.at[pl.ds(rbase, _RPT), :],
                        part_hbm.at[c, pl.ds(rbase, _RPT), :])
        if with_counts:
            pltpu.sync_copy(dacc.at[pl.ds(rbase, _RPT)],
                            dcnt_hbm.at[c, pl.ds(rbase, _RPT)])
            pltpu.sync_copy(bacc.at[pl.ds(rbase, _RPT)],
                            bcnt_hbm.at[c, pl.ds(rbase, _RPT)])

    return pl.kernel(body, out_type=tuple(outs) if with_counts else outs[0],
                     mesh=_mesh, scratch_types=scratch,
                     compiler_params=pltpu.CompilerParams(
                         use_tc_tiling_on_sc=False))


_sc_pass_counts = _build_sc_pass(True)
_sc_pass = _build_sc_pass(False)


def _leaky(x, a):
    return jnp.where(x >= 0, x, a * x)


def _front_body(x_ref, wih_ref, whh_ref, bih_ref, bhh_ref, w2_ref, b2_ref,
                wh1_ref, o_ref):
    x = x_ref[...]
    h = jnp.zeros((_BN, _F), jnp.float32)
    acc = jnp.zeros((_BN, _F), jnp.float32)
    for t in range(_T):
        xt = x[t]
        ir = xt @ wih_ref[0] + bih_ref[0]
        iz = xt @ wih_ref[1] + bih_ref[1]
        inn = xt @ wih_ref[2] + bih_ref[2]
        hr = h @ whh_ref[0] + bhh_ref[0]
        hz = h @ whh_ref[1] + bhh_ref[1]
        hn = h @ whh_ref[2] + bhh_ref[2]
        r = jax.nn.sigmoid(ir + hr)
        z = jax.nn.sigmoid(iz + hz)
        nn = jnp.tanh(inn + r * hn)
        h = (1.0 - z) * nn + z * h
        acc = acc + h @ w2_ref[t]
    out = _leaky(acc + b2_ref[...], 0.01)
    o_ref[...] = out @ wh1_ref[...]


_front = pl.pallas_call(
    _front_body,
    grid=(_NP // _BN,),
    in_specs=[
        pl.BlockSpec((_T, _BN, 6), lambda i: (0, i, 0)),
        pl.BlockSpec((3, 6, _F), lambda i: (0, 0, 0)),
        pl.BlockSpec((3, _F, _F), lambda i: (0, 0, 0)),
        pl.BlockSpec((3, 1, _F), lambda i: (0, 0, 0)),
        pl.BlockSpec((3, 1, _F), lambda i: (0, 0, 0)),
        pl.BlockSpec((_T, _F, _F), lambda i: (0, 0, 0)),
        pl.BlockSpec((1, _F), lambda i: (0, 0)),
        pl.BlockSpec((_F, _F), lambda i: (0, 0)),
    ],
    out_specs=pl.BlockSpec((_BN, _F), lambda i: (i, 0)),
    out_shape=jax.ShapeDtypeStruct((_NP, _F), jnp.float32),
)


def _combe_body(p_ref, b_ref, o_ref):
    sacc = p_ref[0] + p_ref[1]
    bc = b_ref[0] + b_ref[1]
    binv = jnp.where(bc > 0, 1.0 / bc, 0.0)
    o_ref[...] = binv * sacc


_combE = pl.pallas_call(
    _combe_body,
    grid=(_NP // _RB,),
    in_specs=[
        pl.BlockSpec((_NC, _RB, _F), lambda i: (0, i, 0)),
        pl.BlockSpec((_NC, _RB, 1), lambda i: (0, i, 0)),
    ],
    out_specs=pl.BlockSpec((_RB, _F), lambda i: (i, 0)),
    out_shape=jax.ShapeDtypeStruct((_NP, _F), jnp.float32),
)


def _build_combn(final):
    def body(*refs):
        if final:
            p_ref, d_ref, bh_ref, w_ref, b1_ref, o_ref = refs
        else:
            p_ref, d_ref, bh_ref, w_ref, o_ref = refs
        sacc = p_ref[0] + p_ref[1]
        dc = d_ref[0] + d_ref[1]
        dinv = jnp.where(dc > 0, 1.0 / dc, 0.0)
        xv = _leaky(dinv * sacc + bh_ref[...], 0.2)
        y = xv @ w_ref[...]
        if final:
            y = _leaky(y + b1_ref[...], 0.01)
        o_ref[...] = y

    in_specs = [
        pl.BlockSpec((_NC, _RB, _F), lambda i: (0, i, 0)),
        pl.BlockSpec((_NC, _RB, 1), lambda i: (0, i, 0)),
        pl.BlockSpec((1, _F), lambda i: (0, 0)),
        pl.BlockSpec((_F, _F), lambda i: (0, 0)),
    ]
    if final:
        in_specs.append(pl.BlockSpec((1, _F), lambda i: (0, 0)))
    return pl.pallas_call(
        body,
        grid=(_NP // _RB,),
        in_specs=in_specs,
        out_specs=pl.BlockSpec((_RB, _F), lambda i: (i, 0)),
        out_shape=jax.ShapeDtypeStruct((_NP, _F), jnp.float32),
    )


_combN = _build_combn(False)
_final = _build_combn(True)


def kernel(price_input, e, concept, Wih, Whh, bih, bhh, W2, b2, Wh1, bh1,
           Wh2, bh2, W1, b1):
    f32 = jnp.float32
    node_idx = e[0]
    edge_idx = e[1]
    pad = _NNZ_P - _NNZ
    padv = jnp.full((pad,), _PAD_IDX, jnp.int32)
    node2d = jnp.concatenate([node_idx, padv]).reshape(_CHUNKS, _C)
    edge2d = jnp.concatenate([edge_idx, padv]).reshape(_CHUNKS, _C)
    z2 = jnp.zeros((_RPT, _F), f32)
    z1 = jnp.zeros((_RPT,), f32)
    ones = jnp.ones((_C,), f32)

    xp = jnp.transpose(price_input, (1, 0, 2))
    xp = jnp.pad(xp, ((0, 0), (0, _NP - _N), (0, 0)))
    wih3 = jnp.transpose(Wih.reshape(3, _F, 6), (0, 2, 1))
    whh3 = jnp.transpose(Whh.reshape(3, _F, _F), (0, 2, 1))
    bih3 = bih.reshape(3, 1, _F)
    bhh3 = bhh.reshape(3, 1, _F)
    w2t = jnp.transpose(W2).reshape(_T, _F, _F)
    b2r = b2.reshape(1, _F)
    wh1t = jnp.transpose(Wh1)
    wh2t = jnp.transpose(Wh2)
    w1t = jnp.transpose(W1)
    bh1r = bh1.reshape(1, _F)
    bh2r = bh2.reshape(1, _F)
    b1r = b1.reshape(1, _F)

    xw1 = _front(xp, wih3, whh3, bih3, bhh3, w2t, b2r, wh1t)

    pA1, dcnt, bcnt = _sc_pass_counts(xw1, node2d, edge2d, z2, z1, ones)
    dcnt3 = dcnt.reshape(_NC, _NP, 1)
    bcnt3 = bcnt.reshape(_NC, _NP, 1)

    ef1 = _combE(pA1, bcnt3)
    pB1 = _sc_pass(ef1, edge2d, node2d, z2)
    xw2 = _combN(pB1, dcnt3, bh1r, wh2t)

    pA2 = _sc_pass(xw2, node2d, edge2d, z2)
    ef2 = _combE(pA2, bcnt3)
    pB2 = _sc_pass(ef2, edge2d, node2d, z2)
    out = _final(pB2, dcnt3, bh2r, w1t, b1r)
    return out[:_N]


# speedup vs baseline: 23.4658x; 1.2044x over previous
import functools

import jax
import jax.numpy as jnp
from jax import lax
from jax.experimental import pallas as pl
from jax.experimental.pallas import tpu as pltpu
from jax.experimental.pallas import tpu_sc as plsc

_N = 50000
_F = 32
_T = 20
_NNZ = 1600000
_C = 128
_NC = 2
_NS = 16
_NW = _NC * _NS
_CHUNKS = 12544
_NNZ_P = _CHUNKS * _C
_CPT = _CHUNKS // _NW
_G = 8
_NP = 51200
_RPT = _NP // _NS
_PAD_IDX = _N
_BN = 2048
_RB = 6400

_mesh = plsc.VectorSubcoreMesh(
    core_axis_name="c", subcore_axis_name="s", num_cores=_NC, num_subcores=_NS
)


def _build_sc_pass(with_counts):
    outs = [jax.ShapeDtypeStruct((_NC, _NP, _F), jnp.float32)]
    if with_counts:
        outs += [jax.ShapeDtypeStruct((_NC, _NP), jnp.float32)] * 2
    scratch = [
        pltpu.VMEM_SHARED((_NP, _F), jnp.float32),
        pltpu.VMEM((_G, _C), jnp.int32),
        pltpu.VMEM((_G, _C), jnp.int32),
        pltpu.VMEM((4, _C, _F), jnp.float32),
        pltpu.SemaphoreType.DMA,
        pltpu.SemaphoreType.DMA,
        pltpu.SemaphoreType.DMA,
        pltpu.SemaphoreType.DMA,
        pltpu.SemaphoreType.DMA,
        pltpu.SemaphoreType.DMA,
        pltpu.SemaphoreType.DMA,
        pltpu.SemaphoreType.DMA,
    ]
    if with_counts:
        scratch += [
            pltpu.VMEM_SHARED((_NP,), jnp.float32),
            pltpu.VMEM_SHARED((_NP,), jnp.float32),
            pltpu.VMEM((_C,), jnp.float32),
        ]

    def body(*refs):
        if with_counts:
            (x_hbm, src_hbm, dst_hbm, z2_hbm, z1_hbm, ones_hbm,
             part_hbm, dcnt_hbm, bcnt_hbm,
             acc, src_v, dst_v, rows_v, g0, g1, g2, g3, s0, s1, s2, s3,
             dacc, bacc, ones_v) = refs
        else:
            (x_hbm, src_hbm, dst_hbm, z2_hbm, part_hbm,
             acc, src_v, dst_v, rows_v, g0, g1, g2, g3, s0, s1, s2, s3) = refs
        gsems = (g0, g1, g2, g3)
        ssems = (s0, s1, s2, s3)
        c = lax.axis_index("c")
        s = lax.axis_index("s")
        wid = s * _NC + c
        rbase = s * _RPT
        pltpu.sync_copy(z2_hbm, acc.at[pl.ds(rbase, _RPT), :])
        if with_counts:
            pltpu.sync_copy(z1_hbm, dacc.at[pl.ds(rbase, _RPT)])
            pltpu.sync_copy(z1_hbm, bacc.at[pl.ds(rbase, _RPT)])
            pltpu.sync_copy(ones_hbm, ones_v)
        plsc.subcore_barrier()

        def group(g, carry):
            cbase = wid * _CPT + g * _G
            pltpu.sync_copy(src_hbm.at[pl.ds(cbase, _G), :], src_v)
            pltpu.sync_copy(dst_hbm.at[pl.ds(cbase, _G), :], dst_v)
            dg, ds = {}, {}
            depth = 2
            for j in range(depth):
                b = j % 4
                dg[b] = pltpu.async_copy(x_hbm.at[src_v.at[j]],
                                         rows_v.at[b], gsems[b])
            for j in range(_G):
                b = j % 4
                dg[b].wait()
                ds[b] = pltpu.async_copy(rows_v.at[b], acc.at[dst_v.at[j]],
                                         ssems[b], add=True)
                nj = j + depth
                if nj < _G:
                    nb = nj % 4
                    if nj >= 4:
                        ds[nb].wait()
                    dg[nb] = pltpu.async_copy(x_hbm.at[src_v.at[nj]],
                                              rows_v.at[nb], gsems[nb])
                if with_counts:
                    pltpu.sync_copy(ones_v, dacc.at[src_v.at[j]], add=True)
                    pltpu.sync_copy(ones_v, bacc.at[dst_v.at[j]], add=True)
            for j in range(_G - 4, _G):
                ds[j % 4].wait()
            return carry

        lax.fori_loop(0, _CPT // _G, group, 0)
        plsc.subcore_barrier()
        pltpu.sync_copy(acc.at[pl.ds(rbase, _RPT), :],
                        part_hbm.at[c, pl.ds(rbase, _RPT), :])
        if with_counts:
            pltpu.sync_copy(dacc.at[pl.ds(rbase, _RPT)],
                            dcnt_hbm.at[c, pl.ds(rbase, _RPT)])
            pltpu.sync_copy(bacc.at[pl.ds(rbase, _RPT)],
                            bcnt_hbm.at[c, pl.ds(rbase, _RPT)])

    return pl.kernel(body, out_type=tuple(outs) if with_counts else outs[0],
                     mesh=_mesh, scratch_types=scratch,
                     compiler_params=pltpu.CompilerParams(
                         use_tc_tiling_on_sc=False))


_sc_pass_counts = _build_sc_pass(True)
_sc_pass = _build_sc_pass(False)


def _leaky(x, a):
    return jnp.where(x >= 0, x, a * x)


def _front_body(x_ref, wih_ref, whh_ref, bih_ref, bhh_ref, w2_ref, b2_ref,
                wh1_ref, o_ref):
    x = x_ref[...]
    h = jnp.zeros((_BN, _F), jnp.float32)
    acc = jnp.zeros((_BN, _F), jnp.float32)
    for t in range(_T):
        xt = x[t]
        ir = xt @ wih_ref[0] + bih_ref[0]
        iz = xt @ wih_ref[1] + bih_ref[1]
        inn = xt @ wih_ref[2] + bih_ref[2]
        hr = h @ whh_ref[0] + bhh_ref[0]
        hz = h @ whh_ref[1] + bhh_ref[1]
        hn = h @ whh_ref[2] + bhh_ref[2]
        r = jax.nn.sigmoid(ir + hr)
        z = jax.nn.sigmoid(iz + hz)
        nn = jnp.tanh(inn + r * hn)
        h = (1.0 - z) * nn + z * h
        acc = acc + h @ w2_ref[t]
    out = _leaky(acc + b2_ref[...], 0.01)
    o_ref[...] = out @ wh1_ref[...]


_front = pl.pallas_call(
    _front_body,
    grid=(_NP // _BN,),
    in_specs=[
        pl.BlockSpec((_T, _BN, 6), lambda i: (0, i, 0)),
        pl.BlockSpec((3, 6, _F), lambda i: (0, 0, 0)),
        pl.BlockSpec((3, _F, _F), lambda i: (0, 0, 0)),
        pl.BlockSpec((3, 1, _F), lambda i: (0, 0, 0)),
        pl.BlockSpec((3, 1, _F), lambda i: (0, 0, 0)),
        pl.BlockSpec((_T, _F, _F), lambda i: (0, 0, 0)),
        pl.BlockSpec((1, _F), lambda i: (0, 0)),
        pl.BlockSpec((_F, _F), lambda i: (0, 0)),
    ],
    out_specs=pl.BlockSpec((_BN, _F), lambda i: (i, 0)),
    out_shape=jax.ShapeDtypeStruct((_NP, _F), jnp.float32),
)


def _combe_body(p_ref, b_ref, o_ref):
    sacc = p_ref[0] + p_ref[1]
    bc = b_ref[0] + b_ref[1]
    binv = jnp.where(bc > 0, 1.0 / bc, 0.0)
    o_ref[...] = binv * sacc


_combE = pl.pallas_call(
    _combe_body,
    grid=(_NP // _RB,),
    in_specs=[
        pl.BlockSpec((_NC, _RB, _F), lambda i: (0, i, 0)),
        pl.BlockSpec((_NC, _RB, 1), lambda i: (0, i, 0)),
    ],
    out_specs=pl.BlockSpec((_RB, _F), lambda i: (i, 0)),
    out_shape=jax.ShapeDtypeStruct((_NP, _F), jnp.float32),
)


def _build_combn(final):
    def body(*refs):
        if final:
            p_ref, d_ref, bh_ref, w_ref, b1_ref, o_ref = refs
        else:
            p_ref, d_ref, bh_ref, w_ref, o_ref = refs
        sacc = p_ref[0] + p_ref[1]
        dc = d_ref[0] + d_ref[1]
        dinv = jnp.where(dc > 0, 1.0 / dc, 0.0)
        xv = _leaky(dinv * sacc + bh_ref[...], 0.2)
        y = xv @ w_ref[...]
        if final:
            y = _leaky(y + b1_ref[...], 0.01)
        o_ref[...] = y

    in_specs = [
        pl.BlockSpec((_NC, _RB, _F), lambda i: (0, i, 0)),
        pl.BlockSpec((_NC, _RB, 1), lambda i: (0, i, 0)),
        pl.BlockSpec((1, _F), lambda i: (0, 0)),
        pl.BlockSpec((_F, _F), lambda i: (0, 0)),
    ]
    if final:
        in_specs.append(pl.BlockSpec((1, _F), lambda i: (0, 0)))
    return pl.pallas_call(
        body,
        grid=(_NP // _RB,),
        in_specs=in_specs,
        out_specs=pl.BlockSpec((_RB, _F), lambda i: (i, 0)),
        out_shape=jax.ShapeDtypeStruct((_NP, _F), jnp.float32),
    )


_combN = _build_combn(False)
_final = _build_combn(True)


def kernel(price_input, e, concept, Wih, Whh, bih, bhh, W2, b2, Wh1, bh1,
           Wh2, bh2, W1, b1):
    f32 = jnp.float32
    node_idx = e[0]
    edge_idx = e[1]
    pad = _NNZ_P - _NNZ
    padv = jnp.full((pad,), _PAD_IDX, jnp.int32)
    node2d = jnp.concatenate([node_idx, padv]).reshape(_CHUNKS, _C)
    edge2d = jnp.concatenate([edge_idx, padv]).reshape(_CHUNKS, _C)
    z2 = jnp.zeros((_RPT, _F), f32)
    z1 = jnp.zeros((_RPT,), f32)
    ones = jnp.ones((_C,), f32)

    xp = jnp.transpose(price_input, (1, 0, 2))
    xp = jnp.pad(xp, ((0, 0), (0, _NP - _N), (0, 0)))
    wih3 = jnp.transpose(Wih.reshape(3, _F, 6), (0, 2, 1))
    whh3 = jnp.transpose(Whh.reshape(3, _F, _F), (0, 2, 1))
    bih3 = bih.reshape(3, 1, _F)
    bhh3 = bhh.reshape(3, 1, _F)
    w2t = jnp.transpose(W2).reshape(_T, _F, _F)
    b2r = b2.reshape(1, _F)
    wh1t = jnp.transpose(Wh1)
    wh2t = jnp.transpose(Wh2)
    w1t = jnp.transpose(W1)
    bh1r = bh1.reshape(1, _F)
    bh2r = bh2.reshape(1, _F)
    b1r = b1.reshape(1, _F)

    xw1 = _front(xp, wih3, whh3, bih3, bhh3, w2t, b2r, wh1t)

    pA1, dcnt, bcnt = _sc_pass_counts(xw1, node2d, edge2d, z2, z1, ones)
    dcnt3 = dcnt.reshape(_NC, _NP, 1)
    bcnt3 = bcnt.reshape(_NC, _NP, 1)

    ef1 = _combE(pA1, bcnt3)
    pB1 = _sc_pass(ef1, edge2d, node2d, z2)
    xw2 = _combN(pB1, dcnt3, bh1r, wh2t)

    pA2 = _sc_pass(xw2, node2d, edge2d, z2)
    ef2 = _combE(pA2, bcnt3)
    pB2 = _sc_pass(ef2, edge2d, node2d, z2)
    out = _final(pB2, dcnt3, bh2r, w1t, b1r)
    return out[:_N]


# speedup vs baseline: 37.6183x; 1.6031x over previous
import functools

import jax
import jax.numpy as jnp
from jax import lax
from jax.experimental import pallas as pl
from jax.experimental.pallas import tpu as pltpu
from jax.experimental.pallas import tpu_sc as plsc

_N = 50000
_F = 32
_T = 20
_NNZ = 1600000
_C = 128
_NC = 2
_NS = 16
_NW = _NC * _NS
_CHUNKS = _NNZ // _C
_WBASE = _CHUNKS // _NW
_WEXTRA = _CHUNKS % _NW
_FULLG = _WBASE // 8
_G = 8
_NP = 50176
_RPT = _NP // _NS
_BN = 2000
_RB = 6272

_mesh = plsc.VectorSubcoreMesh(
    core_axis_name="c", subcore_axis_name="s", num_cores=_NC, num_subcores=_NS
)


def _build_sc_pass(with_counts):
    nbuf = 5 if with_counts else 6
    depth = 3 if with_counts else 4
    outs = [jax.ShapeDtypeStruct((_NC, _NP, _F), jnp.float32)]
    if with_counts:
        outs += [jax.ShapeDtypeStruct((_NC, _NP), jnp.float32)] * 2
    scratch = [
        pltpu.VMEM_SHARED((_NP, _F), jnp.float32),
        pltpu.VMEM((_G, _C), jnp.int32),
        pltpu.VMEM((_G, _C), jnp.int32),
        pltpu.VMEM((nbuf, _C, _F), jnp.float32),
        pltpu.SemaphoreType.DMA((nbuf,)),
        pltpu.SemaphoreType.DMA((nbuf,)),
    ]
    if with_counts:
        scratch += [
            pltpu.VMEM_SHARED((_NP,), jnp.float32),
            pltpu.VMEM_SHARED((_NP,), jnp.float32),
            pltpu.VMEM((_C,), jnp.float32),
        ]

    def body(*refs):
        if with_counts:
            (x_hbm, src_hbm, dst_hbm, z2_hbm, z1_hbm, ones_hbm,
             part_hbm, dcnt_hbm, bcnt_hbm,
             acc, src_v, dst_v, rows_v, gsem, ssem,
             dacc, bacc, ones_v) = refs
        else:
            (x_hbm, src_hbm, dst_hbm, z2_hbm, part_hbm,
             acc, src_v, dst_v, rows_v, gsem, ssem) = refs
        c = lax.axis_index("c")
        s = lax.axis_index("s")
        wid = s * _NC + c
        rbase = s * _RPT
        base = wid * _WBASE + jnp.minimum(wid, _WEXTRA)
        pltpu.sync_copy(z2_hbm, acc.at[pl.ds(rbase, _RPT), :])
        if with_counts:
            pltpu.sync_copy(z1_hbm, dacc.at[pl.ds(rbase, _RPT)])
            pltpu.sync_copy(z1_hbm, bacc.at[pl.ds(rbase, _RPT)])
            pltpu.sync_copy(ones_hbm, ones_v)
        plsc.subcore_barrier()

        def group(g, carry):
            cbase = base + g * _G
            pltpu.sync_copy(src_hbm.at[pl.ds(cbase, _G), :], src_v)
            pltpu.sync_copy(dst_hbm.at[pl.ds(cbase, _G), :], dst_v)
            dg, ds = {}, {}
            for j in range(depth):
                b = j % nbuf
                dg[j] = pltpu.async_copy(x_hbm.at[src_v.at[j]],
                                         rows_v.at[b], gsem.at[b])
            for j in range(_G):
                b = j % nbuf
                dg[j].wait()
                ds[j] = pltpu.async_copy(rows_v.at[b], acc.at[dst_v.at[j]],
                                         ssem.at[b], add=True)
                nj = j + depth
                if nj < _G:
                    nb = nj % nbuf
                    if nj - nbuf >= 0:
                        ds[nj - nbuf].wait()
                    dg[nj] = pltpu.async_copy(x_hbm.at[src_v.at[nj]],
                                              rows_v.at[nb], gsem.at[nb])
                if with_counts:
                    pltpu.sync_copy(ones_v, dacc.at[src_v.at[j]], add=True)
                    pltpu.sync_copy(ones_v, bacc.at[dst_v.at[j]], add=True)
            for j in range(max(0, _G - nbuf), _G):
                ds[j].wait()
            return carry

        lax.fori_loop(0, _FULLG, group, 0)

        ntail = _WBASE - _FULLG * _G + jnp.where(wid < _WEXTRA, 1, 0)

        def tail(j, carry):
            cid = base + _FULLG * _G + j
            pltpu.sync_copy(src_hbm.at[pl.ds(cid, 1), :],
                            src_v.at[pl.ds(0, 1), :])
            pltpu.sync_copy(dst_hbm.at[pl.ds(cid, 1), :],
                            dst_v.at[pl.ds(0, 1), :])
            pltpu.async_copy(x_hbm.at[src_v.at[0]], rows_v.at[0],
                             gsem.at[0]).wait()
            pltpu.sync_copy(rows_v.at[0], acc.at[dst_v.at[0]], add=True)
            if with_counts:
                pltpu.sync_copy(ones_v, dacc.at[src_v.at[0]], add=True)
                pltpu.sync_copy(ones_v, bacc.at[dst_v.at[0]], add=True)
            return carry

        lax.fori_loop(0, ntail, tail, 0)
        plsc.subcore_barrier()
        pltpu.sync_copy(acc.at[pl.ds(rbase, _RPT), :],
                        part_hbm.at[c, pl.ds(rbase, _RPT), :])
        if with_counts:
            pltpu.sync_copy(dacc.at[pl.ds(rbase, _RPT)],
                            dcnt_hbm.at[c, pl.ds(rbase, _RPT)])
            pltpu.sync_copy(bacc.at[pl.ds(rbase, _RPT)],
                            bcnt_hbm.at[c, pl.ds(rbase, _RPT)])

    return pl.kernel(body, out_type=tuple(outs) if with_counts else outs[0],
                     mesh=_mesh, scratch_types=scratch,
                     compiler_params=pltpu.CompilerParams(
                         use_tc_tiling_on_sc=False))


_sc_pass_counts = _build_sc_pass(True)
_sc_pass = _build_sc_pass(False)


def _leaky(x, a):
    return jnp.where(x >= 0, x, a * x)


def _front_body(x_ref, wb_ref, whh_ref, bih_ref, bhh_ref, w2_ref, b2_ref,
                wh1_ref, o_ref):
    xb = x_ref[...]
    gi = lax.dot_general(wb_ref[...], xb, (((1,), (1,)), ((), ())),
                         preferred_element_type=jnp.float32)
    whh = whh_ref[...]
    bhh = bhh_ref[...]
    bih = bih_ref[...]
    h = jnp.zeros((_F, _BN), jnp.float32)
    acc = jnp.zeros((_F, _BN), jnp.float32)
    for t in range(_T):
        git = gi[96 * t:96 * (t + 1)] + bih
        gh = jnp.dot(whh, h, preferred_element_type=jnp.float32) + bhh
        r = jax.nn.sigmoid(git[0:32] + gh[0:32])
        z = jax.nn.sigmoid(git[32:64] + gh[32:64])
        nn = jnp.tanh(git[64:96] + r * gh[64:96])
        h = (1.0 - z) * nn + z * h
        acc = acc + jnp.dot(w2_ref[t], h, preferred_element_type=jnp.float32)
    out = _leaky(acc + b2_ref[...], 0.01)
    o_ref[...] = lax.dot_general(out, wh1_ref[...], (((0,), (1,)), ((), ())),
                                 preferred_element_type=jnp.float32)


_front = pl.pallas_call(
    _front_body,
    grid=(_N // _BN,),
    in_specs=[
        pl.BlockSpec((_BN, _T * 6), lambda i: (i, 0)),
        pl.BlockSpec((_T * 96, _T * 6), lambda i: (0, 0)),
        pl.BlockSpec((96, _F), lambda i: (0, 0)),
        pl.BlockSpec((96, 1), lambda i: (0, 0)),
        pl.BlockSpec((96, 1), lambda i: (0, 0)),
        pl.BlockSpec((_T, _F, _F), lambda i: (0, 0, 0)),
        pl.BlockSpec((_F, 1), lambda i: (0, 0)),
        pl.BlockSpec((_F, _F), lambda i: (0, 0)),
    ],
    out_specs=pl.BlockSpec((_BN, _F), lambda i: (i, 0)),
    out_shape=jax.ShapeDtypeStruct((_N, _F), jnp.float32),
)


def _combe_body(p_ref, b_ref, o_ref):
    sacc = p_ref[0] + p_ref[1]
    bc = b_ref[0] + b_ref[1]
    binv = jnp.where(bc > 0, 1.0 / bc, 0.0)
    o_ref[...] = binv * sacc


_combE = pl.pallas_call(
    _combe_body,
    grid=(_NP // _RB,),
    in_specs=[
        pl.BlockSpec((_NC, _RB, _F), lambda i: (0, i, 0)),
        pl.BlockSpec((_NC, _RB, 1), lambda i: (0, i, 0)),
    ],
    out_specs=pl.BlockSpec((_RB, _F), lambda i: (i, 0)),
    out_shape=jax.ShapeDtypeStruct((_NP, _F), jnp.float32),
)


def _build_combn(final):
    def body(*refs):
        if final:
            p_ref, d_ref, bh_ref, w_ref, b1_ref, o_ref = refs
        else:
            p_ref, d_ref, bh_ref, w_ref, o_ref = refs
        sacc = p_ref[0] + p_ref[1]
        dc = d_ref[0] + d_ref[1]
        dinv = jnp.where(dc > 0, 1.0 / dc, 0.0)
        xv = _leaky(dinv * sacc + bh_ref[...], 0.2)
        y = xv @ w_ref[...]
        if final:
            y = _leaky(y + b1_ref[...], 0.01)
        o_ref[...] = y

    in_specs = [
        pl.BlockSpec((_NC, _RB, _F), lambda i: (0, i, 0)),
        pl.BlockSpec((_NC, _RB, 1), lambda i: (0, i, 0)),
        pl.BlockSpec((1, _F), lambda i: (0, 0)),
        pl.BlockSpec((_F, _F), lambda i: (0, 0)),
    ]
    if final:
        in_specs.append(pl.BlockSpec((1, _F), lambda i: (0, 0)))
    return pl.pallas_call(
        body,
        grid=(_NP // _RB,),
        in_specs=in_specs,
        out_specs=pl.BlockSpec((_RB, _F), lambda i: (i, 0)),
        out_shape=jax.ShapeDtypeStruct((_NP, _F), jnp.float32),
    )


_combN = _build_combn(False)
_final = _build_combn(True)


def kernel(price_input, e, concept, Wih, Whh, bih, bhh, W2, b2, Wh1, bh1,
           Wh2, bh2, W1, b1):
    f32 = jnp.float32
    node2d = e[0].reshape(_CHUNKS, _C)
    edge2d = e[1].reshape(_CHUNKS, _C)
    z2 = jnp.zeros((_RPT, _F), f32)
    z1 = jnp.zeros((_RPT,), f32)
    ones = jnp.ones((_C,), f32)

    x2d = price_input.reshape(_N, _T * 6)
    wblk = jnp.kron(jnp.eye(_T, dtype=f32), Wih)
    bih2 = bih.reshape(96, 1)
    bhh2 = bhh.reshape(96, 1)
    w2t = jnp.transpose(W2.reshape(_F, _T, _F), (1, 0, 2))
    b2r = b2.reshape(_F, 1)
    wh2t = jnp.transpose(Wh2)
    w1t = jnp.transpose(W1)
    bh1r = bh1.reshape(1, _F)
    bh2r = bh2.reshape(1, _F)
    b1r = b1.reshape(1, _F)

    xw1 = _front(x2d, wblk, Whh, bih2, bhh2, w2t, b2r, Wh1)

    pA1, dcnt, bcnt = _sc_pass_counts(xw1, node2d, edge2d, z2, z1, ones)
    dcnt3 = dcnt.reshape(_NC, _NP, 1)
    bcnt3 = bcnt.reshape(_NC, _NP, 1)

    ef1 = _combE(pA1, bcnt3)
    pB1 = _sc_pass(ef1, edge2d, node2d, z2)
    xw2 = _combN(pB1, dcnt3, bh1r, wh2t)

    pA2 = _sc_pass(xw2, node2d, edge2d, z2)
    ef2 = _combE(pA2, bcnt3)
    pB2 = _sc_pass(ef2, edge2d, node2d, z2)
    out = _final(pB2, dcnt3, bh2r, w1t, b1r)
    return out[:_N]


# speedup vs baseline: 45.0815x; 1.1984x over previous
import functools

import jax
import jax.numpy as jnp
from jax import lax
from jax.experimental import pallas as pl
from jax.experimental.pallas import tpu as pltpu
from jax.experimental.pallas import tpu_sc as plsc

_N = 50000
_F = 32
_T = 20
_NNZ = 1600000
_C = 128
_NC = 2
_NS = 16
_NW = _NC * _NS
_CHUNKS = _NNZ // _C
_WBASE = _CHUNKS // _NW
_WEXTRA = _CHUNKS % _NW
_FULLG = _WBASE // 8
_G = 8
_NP = 50176
_RPT = _NP // _NS
_BN = 2000
_RB = 6272
_NBUF = 6
_DEPTH = 4

_mesh = plsc.VectorSubcoreMesh(
    core_axis_name="c", subcore_axis_name="s", num_cores=_NC, num_subcores=_NS
)


def _worker_base(wid):
    return wid * _WBASE + jnp.minimum(wid, _WEXTRA)


def _build_sc_pass(si, di):
    scratch = [
        pltpu.VMEM_SHARED((_NP, _F), jnp.float32),
        pltpu.VMEM((2, _G, _C), jnp.int32),
        pltpu.VMEM((2, _G, _C), jnp.int32),
        pltpu.VMEM((_NBUF, _C, _F), jnp.float32),
        pltpu.SemaphoreType.DMA((_NBUF,)),
        pltpu.SemaphoreType.DMA((_NBUF,)),
        pltpu.SemaphoreType.DMA((2,)),
        pltpu.SemaphoreType.DMA((2,)),
    ]

    def body(x_hbm, e_hbm, z2_hbm, part_hbm,
             acc, src_v, dst_v, rows_v, gsem, ssem, isem, jsem):
        c = lax.axis_index("c")
        s = lax.axis_index("s")
        wid = s * _NC + c
        rbase = s * _RPT
        base = _worker_base(wid)
        pltpu.sync_copy(z2_hbm, acc.at[pl.ds(rbase, _RPT), :])

        def start_idx(g, slot):
            pltpu.async_copy(e_hbm.at[si, pl.ds(base + g * _G, _G), :],
                             src_v.at[slot], isem.at[slot])
            pltpu.async_copy(e_hbm.at[di, pl.ds(base + g * _G, _G), :],
                             dst_v.at[slot], jsem.at[slot])

        def wait_idx(slot):
            pltpu.make_async_copy(e_hbm.at[si, pl.ds(0, _G), :],
                                  src_v.at[slot], isem.at[slot]).wait()
            pltpu.make_async_copy(e_hbm.at[di, pl.ds(0, _G), :],
                                  dst_v.at[slot], jsem.at[slot]).wait()

        start_idx(0, 0)
        plsc.subcore_barrier()

        def chunks(slot):
            dg, ds = {}, {}
            for j in range(_DEPTH):
                b = j % _NBUF
                dg[j] = pltpu.async_copy(x_hbm.at[src_v.at[slot, j]],
                                         rows_v.at[b], gsem.at[b])
            for j in range(_G):
                b = j % _NBUF
                dg[j].wait()
                ds[j] = pltpu.async_copy(rows_v.at[b],
                                         acc.at[dst_v.at[slot, j]],
                                         ssem.at[b], add=True)
                nj = j + _DEPTH
                if nj < _G:
                    nb = nj % _NBUF
                    if nj - _NBUF >= 0:
                        ds[nj - _NBUF].wait()
                    dg[nj] = pltpu.async_copy(x_hbm.at[src_v.at[slot, nj]],
                                              rows_v.at[nb], gsem.at[nb])
            for j in range(max(0, _G - _NBUF), _G):
                ds[j].wait()

        def dgroup(k, carry):
            g = k * 2
            wait_idx(0)
            start_idx(g + 1, 1)
            chunks(0)
            wait_idx(1)
            start_idx(g + 2, 0)
            chunks(1)
            return carry

        lax.fori_loop(0, (_FULLG - 1) // 2, dgroup, 0)
        wait_idx(0)
        chunks(0)

        ntail = _WBASE - _FULLG * _G + jnp.where(wid < _WEXTRA, 1, 0)

        def tail(j, carry):
            cid = base + _FULLG * _G + j
            pltpu.sync_copy(e_hbm.at[si, pl.ds(cid, 1), :],
                            src_v.at[0, pl.ds(0, 1), :])
            pltpu.sync_copy(e_hbm.at[di, pl.ds(cid, 1), :],
                            dst_v.at[0, pl.ds(0, 1), :])
            pltpu.async_copy(x_hbm.at[src_v.at[0, 0]], rows_v.at[0],
                             gsem.at[0]).wait()
            pltpu.sync_copy(rows_v.at[0], acc.at[dst_v.at[0, 0]], add=True)
            return carry

        lax.fori_loop(0, ntail, tail, 0)
        plsc.subcore_barrier()
        pltpu.sync_copy(acc.at[pl.ds(rbase, _RPT), :],
                        part_hbm.at[c, pl.ds(rbase, _RPT), :])

    return pl.kernel(body,
                     out_type=jax.ShapeDtypeStruct((_NC, _NP, _F),
                                                   jnp.float32),
                     mesh=_mesh, scratch_types=scratch,
                     compiler_params=pltpu.CompilerParams(
                         use_tc_tiling_on_sc=False))


_sc_pass_ne = _build_sc_pass(0, 1)
_sc_pass_en = _build_sc_pass(1, 0)


def _counts_body(e_hbm, z1_hbm, ones_hbm, dcnt_hbm, bcnt_hbm,
                 dacc, bacc, src_v, dst_v, ones_v):
    c = lax.axis_index("c")
    s = lax.axis_index("s")
    wid = s * _NC + c
    rbase = s * _RPT
    base = _worker_base(wid)
    pltpu.sync_copy(z1_hbm, dacc.at[pl.ds(rbase, _RPT)])
    pltpu.sync_copy(z1_hbm, bacc.at[pl.ds(rbase, _RPT)])
    pltpu.sync_copy(ones_hbm, ones_v)
    plsc.subcore_barrier()

    def group(g, carry):
        cbase = base + g * _G
        pltpu.sync_copy(e_hbm.at[0, pl.ds(cbase, _G), :], src_v)
        pltpu.sync_copy(e_hbm.at[1, pl.ds(cbase, _G), :], dst_v)
        for j in range(_G):
            pltpu.sync_copy(ones_v, dacc.at[src_v.at[j]], add=True)
            pltpu.sync_copy(ones_v, bacc.at[dst_v.at[j]], add=True)
        return carry

    lax.fori_loop(0, _FULLG, group, 0)
    ntail = _WBASE - _FULLG * _G + jnp.where(wid < _WEXTRA, 1, 0)

    def tail(j, carry):
        cid = base + _FULLG * _G + j
        pltpu.sync_copy(e_hbm.at[0, pl.ds(cid, 1), :],
                        src_v.at[pl.ds(0, 1), :])
        pltpu.sync_copy(e_hbm.at[1, pl.ds(cid, 1), :],
                        dst_v.at[pl.ds(0, 1), :])
        pltpu.sync_copy(ones_v, dacc.at[src_v.at[0]], add=True)
        pltpu.sync_copy(ones_v, bacc.at[dst_v.at[0]], add=True)
        return carry

    lax.fori_loop(0, ntail, tail, 0)
    plsc.subcore_barrier()
    pltpu.sync_copy(dacc.at[pl.ds(rbase, _RPT)],
                    dcnt_hbm.at[c, pl.ds(rbase, _RPT)])
    pltpu.sync_copy(bacc.at[pl.ds(rbase, _RPT)],
                    bcnt_hbm.at[c, pl.ds(rbase, _RPT)])


_sc_counts = pl.kernel(
    _counts_body,
    out_type=(jax.ShapeDtypeStruct((_NC, _NP), jnp.float32),
              jax.ShapeDtypeStruct((_NC, _NP), jnp.float32)),
    mesh=_mesh,
    scratch_types=[
        pltpu.VMEM_SHARED((_NP,), jnp.float32),
        pltpu.VMEM_SHARED((_NP,), jnp.float32),
        pltpu.VMEM((_G, _C), jnp.int32),
        pltpu.VMEM((_G, _C), jnp.int32),
        pltpu.VMEM((_C,), jnp.float32),
    ],
    compiler_params=pltpu.CompilerParams(use_tc_tiling_on_sc=False),
)


def _leaky(x, a):
    return jnp.where(x >= 0, x, a * x)


def _front_body(x_ref, wb_ref, whh_ref, bih_ref, bhh_ref, w2_ref, b2_ref,
                wh1_ref, o_ref):
    xb = x_ref[...]
    gi = lax.dot_general(wb_ref[...], xb, (((1,), (1,)), ((), ())),
                         preferred_element_type=jnp.float32)
    whh = whh_ref[...]
    bhh = bhh_ref[...]
    bih = bih_ref[...]
    h = jnp.zeros((_F, _BN), jnp.float32)
    acc = jnp.zeros((_F, _BN), jnp.float32)
    for t in range(_T):
        git = gi[96 * t:96 * (t + 1)] + bih
        gh = jnp.dot(whh, h, preferred_element_type=jnp.float32) + bhh
        r = jax.nn.sigmoid(git[0:32] + gh[0:32])
        z = jax.nn.sigmoid(git[32:64] + gh[32:64])
        nn = jnp.tanh(git[64:96] + r * gh[64:96])
        h = (1.0 - z) * nn + z * h
        acc = acc + jnp.dot(w2_ref[t], h, preferred_element_type=jnp.float32)
    out = _leaky(acc + b2_ref[...], 0.01)
    o_ref[...] = lax.dot_general(out, wh1_ref[...], (((0,), (1,)), ((), ())),
                                 preferred_element_type=jnp.float32)


_front = pl.pallas_call(
    _front_body,
    grid=(_N // _BN,),
    in_specs=[
        pl.BlockSpec((_BN, _T * 6), lambda i: (i, 0)),
        pl.BlockSpec((_T * 96, _T * 6), lambda i: (0, 0)),
        pl.BlockSpec((96, _F), lambda i: (0, 0)),
        pl.BlockSpec((96, 1), lambda i: (0, 0)),
        pl.BlockSpec((96, 1), lambda i: (0, 0)),
        pl.BlockSpec((_T, _F, _F), lambda i: (0, 0, 0)),
        pl.BlockSpec((_F, 1), lambda i: (0, 0)),
        pl.BlockSpec((_F, _F), lambda i: (0, 0)),
    ],
    out_specs=pl.BlockSpec((_BN, _F), lambda i: (i, 0)),
    out_shape=jax.ShapeDtypeStruct((_N, _F), jnp.float32),
)


def _combe_body(p_ref, b_ref, o_ref):
    sacc = p_ref[0] + p_ref[1]
    bc = b_ref[0] + b_ref[1]
    binv = jnp.where(bc > 0, 1.0 / bc, 0.0)
    o_ref[...] = binv * sacc


_combE = pl.pallas_call(
    _combe_body,
    grid=(_NP // _RB,),
    in_specs=[
        pl.BlockSpec((_NC, _RB, _F), lambda i: (0, i, 0)),
        pl.BlockSpec((_NC, _RB, 1), lambda i: (0, i, 0)),
    ],
    out_specs=pl.BlockSpec((_RB, _F), lambda i: (i, 0)),
    out_shape=jax.ShapeDtypeStruct((_NP, _F), jnp.float32),
)


def _build_combn(final):
    def body(*refs):
        if final:
            p_ref, d_ref, bh_ref, w_ref, b1_ref, o_ref = refs
        else:
            p_ref, d_ref, bh_ref, w_ref, o_ref = refs
        sacc = p_ref[0] + p_ref[1]
        dc = d_ref[0] + d_ref[1]
        dinv = jnp.where(dc > 0, 1.0 / dc, 0.0)
        xv = _leaky(dinv * sacc + bh_ref[...], 0.2)
        y = xv @ w_ref[...]
        if final:
            y = _leaky(y + b1_ref[...], 0.01)
        o_ref[...] = y

    in_specs = [
        pl.BlockSpec((_NC, _RB, _F), lambda i: (0, i, 0)),
        pl.BlockSpec((_NC, _RB, 1), lambda i: (0, i, 0)),
        pl.BlockSpec((1, _F), lambda i: (0, 0)),
        pl.BlockSpec((_F, _F), lambda i: (0, 0)),
    ]
    if final:
        in_specs.append(pl.BlockSpec((1, _F), lambda i: (0, 0)))
    return pl.pallas_call(
        body,
        grid=(_NP // _RB,),
        in_specs=in_specs,
        out_specs=pl.BlockSpec((_RB, _F), lambda i: (i, 0)),
        out_shape=jax.ShapeDtypeStruct((_NP, _F), jnp.float32),
    )


_combN = _build_combn(False)
_final = _build_combn(True)


def kernel(price_input, e, concept, Wih, Whh, bih, bhh, W2, b2, Wh1, bh1,
           Wh2, bh2, W1, b1):
    f32 = jnp.float32
    e3 = e.reshape(2, _CHUNKS, _C)
    z2 = jnp.zeros((_RPT, _F), f32)
    z1 = jnp.zeros((_RPT,), f32)
    ones = jnp.ones((_C,), f32)

    x2d = price_input.reshape(_N, _T * 6)
    wblk = jnp.kron(jnp.eye(_T, dtype=f32), Wih)
    bih2 = bih.reshape(96, 1)
    bhh2 = bhh.reshape(96, 1)
    w2t = jnp.transpose(W2.reshape(_F, _T, _F), (1, 0, 2))
    b2r = b2.reshape(_F, 1)
    wh2t = jnp.transpose(Wh2)
    w1t = jnp.transpose(W1)
    bh1r = bh1.reshape(1, _F)
    bh2r = bh2.reshape(1, _F)
    b1r = b1.reshape(1, _F)

    dcnt, bcnt = _sc_counts(e3, z1, ones)
    xw1 = _front(x2d, wblk, Whh, bih2, bhh2, w2t, b2r, Wh1)
    dcnt3 = dcnt.reshape(_NC, _NP, 1)
    bcnt3 = bcnt.reshape(_NC, _NP, 1)

    pA1 = _sc_pass_ne(xw1, e3, z2)
    ef1 = _combE(pA1, bcnt3)
    pB1 = _sc_pass_en(ef1, e3, z2)
    xw2 = _combN(pB1, dcnt3, bh1r, wh2t)

    pA2 = _sc_pass_ne(xw2, e3, z2)
    ef2 = _combE(pA2, bcnt3)
    pB2 = _sc_pass_en(ef2, e3, z2)
    out = _final(pB2, dcnt3, bh2r, w1t, b1r)
    return out[:_N]
